# Initial kernel scaffold; baseline (speedup 1.0000x reference)
import jax
import jax.numpy as jnp
from jax.experimental import pallas as pl


def kernel(qc_flags, table):
    raise NotImplementedError("write your pallas kernel here")


# speedup vs baseline: 1.6879x; 1.6879x over previous
import functools

import jax
import jax.numpy as jnp
from jax import lax
from jax.experimental import pallas as pl
from jax.experimental.pallas import tpu as pltpu
from jax.experimental.pallas import tpu_sc as plsc

NUM_BITS = 10
EMBED_DIM = 32
ROWS = 16384
COLS = 100
N_FLAGS = ROWS * COLS

NC = 2
NS = 16
L = 16
NW = NC * NS
B_PER_W = N_FLAGS // NW
CH = 2048
N_CHUNKS = B_PER_W // CH


def _sc_body(flags_hbm, table_hbm, valid_hbm, emb_hbm, bits_hbm,
             idx_v, rows_v, bits_v, valid_v, sem):
    wid = lax.axis_index("s") * NC + lax.axis_index("c")

    def chunk(ci, carry):
        base = wid * B_PER_W + ci * CH
        pltpu.sync_copy(flags_hbm.at[pl.ds(base, CH)], idx_v)
        gather = pltpu.async_copy(table_hbm.at[idx_v], rows_v, sem)

        def vec(vi, c2):
            f = idx_v[pl.ds(vi * L, L)]
            valid_v[pl.ds(vi * L, L)] = jnp.where(
                f == 0, jnp.float32(1.0), jnp.float32(0.0))
            for j in range(NUM_BITS):
                pos = lax.iota(jnp.int32, L) + j * L
                fl = pos // NUM_BITS
                b = pos - fl * NUM_BITS
                fj = lax.gather(
                    f, fl[:, None],
                    lax.GatherDimensionNumbers(
                        offset_dims=(), collapsed_slice_dims=(0,),
                        start_index_map=(0,)),
                    slice_sizes=(1,),
                    mode=lax.GatherScatterMode.PROMISE_IN_BOUNDS)
                bit = ((fj >> b) & 1).astype(jnp.float32)
                bits_v[pl.ds((vi * NUM_BITS + j) * L, L)] = bit
            return c2

        lax.fori_loop(0, CH // L, vec, 0)
        gather.wait()
        pltpu.sync_copy(rows_v, emb_hbm.at[pl.ds(base, CH)])
        pltpu.sync_copy(bits_v, bits_hbm.at[pl.ds(base * NUM_BITS, CH * NUM_BITS)])
        pltpu.sync_copy(valid_v, valid_hbm.at[pl.ds(base, CH)])
        return carry

    lax.fori_loop(0, N_CHUNKS, chunk, 0)


@jax.jit
def _featurize(flags, table):
    mesh = plsc.VectorSubcoreMesh(core_axis_name="c", subcore_axis_name="s")
    fn = functools.partial(
        pl.kernel,
        out_type=(
            jax.ShapeDtypeStruct((N_FLAGS,), jnp.float32),
            jax.ShapeDtypeStruct((N_FLAGS, EMBED_DIM), jnp.float32),
            jax.ShapeDtypeStruct((N_FLAGS * NUM_BITS,), jnp.float32),
        ),
        mesh=mesh,
        compiler_params=pltpu.CompilerParams(
            needs_layout_passes=False, use_tc_tiling_on_sc=False),
        scratch_types=[
            pltpu.VMEM((CH,), jnp.int32),
            pltpu.VMEM((CH, EMBED_DIM), jnp.float32),
            pltpu.VMEM((CH * NUM_BITS,), jnp.float32),
            pltpu.VMEM((CH,), jnp.float32),
            pltpu.SemaphoreType.DMA,
        ],
    )(_sc_body)
    return fn(flags, table)


def kernel(qc_flags, table):
    flags = qc_flags.reshape(-1).astype(jnp.int32)
    valid, emb, bits = _featurize(flags, table)
    return (
        valid.reshape(ROWS, COLS),
        emb.reshape(ROWS, COLS, EMBED_DIM),
        bits.reshape(ROWS, COLS, NUM_BITS),
    )


# speedup vs baseline: 13.8616x; 8.2123x over previous
import functools

import jax
import jax.numpy as jnp
from jax import lax
from jax.experimental import pallas as pl
from jax.experimental.pallas import tpu as pltpu
from jax.experimental.pallas import tpu_sc as plsc

NUM_BITS = 10
EMBED_DIM = 32
ROWS = 16384
COLS = 100
VOCAB = 1024

NC = 2
NS = 16
L = 16
NW = NC * NS
RW = ROWS // NW
CC = 20
NCH = COLS // CC


def _sc_body(flagsT, tableT, validT, embT, bitsT,
             table_v, flags_v, emb_v, bits_v, valid_v,
             in_sem, out_sem0, out_sem1):
    wid = lax.axis_index("s") * NC + lax.axis_index("c")
    r0 = wid * RW
    out_sems = (out_sem0, out_sem1)

    pltpu.sync_copy(tableT, table_v)
    pltpu.async_copy(
        flagsT.at[pl.ds(0, CC), pl.ds(r0, RW)], flags_v.at[0], in_sem
    ).wait()

    for k in range(NCH):
        in_slot = k % 2
        if k + 1 < NCH:
            pltpu.async_copy(
                flagsT.at[pl.ds((k + 1) * CC, CC), pl.ds(r0, RW)],
                flags_v.at[(k + 1) % 2], in_sem)

        @pl.loop(0, CC, step=2)
        def _col_pair(c2, _k=k, _slot=in_slot):
            for b in range(2):
                cl = c2 + b
                c = _k * CC + cl
                sem = out_sems[b]

                def drain():
                    pltpu.make_async_copy(
                        valid_v.at[b], validT.at[c, pl.ds(r0, RW)], sem
                    ).wait()
                    pltpu.make_async_copy(
                        bits_v.at[b], bitsT.at[:, c, pl.ds(r0, RW)], sem
                    ).wait()
                    pltpu.make_async_copy(
                        emb_v.at[b], embT.at[c, :, pl.ds(r0, RW)], sem
                    ).wait()

                if _k == 0:
                    @pl.when(c2 + b >= 2)
                    def _():
                        drain()
                else:
                    drain()

                @pl.loop(0, RW // L)
                def _vec(vi):
                    f = flags_v[_slot, cl, pl.ds(vi * L, L)]
                    valid_v[b, pl.ds(vi * L, L)] = jnp.where(
                        f == 0, jnp.float32(1.0), jnp.float32(0.0))
                    for t in range(NUM_BITS):
                        bits_v[b, t, pl.ds(vi * L, L)] = (
                            (f >> t) & 1).astype(jnp.float32)
                    for d in range(EMBED_DIM):
                        emb_v[b, d, pl.ds(vi * L, L)] = plsc.load_gather(
                            table_v, [f + d * VOCAB])

                pltpu.async_copy(
                    valid_v.at[b], validT.at[c, pl.ds(r0, RW)], sem)
                pltpu.async_copy(
                    bits_v.at[b], bitsT.at[:, c, pl.ds(r0, RW)], sem)
                pltpu.async_copy(
                    emb_v.at[b], embT.at[c, :, pl.ds(r0, RW)], sem)

        if k > 0 or True:
            pass

        if k + 1 < NCH:
            pltpu.make_async_copy(
                flagsT.at[pl.ds((k + 1) * CC, CC), pl.ds(r0, RW)],
                flags_v.at[(k + 1) % 2], in_sem
            ).wait()

    for b in range(2):
        c = COLS - 2 + b
        sem = out_sems[b]
        pltpu.make_async_copy(
            valid_v.at[b], validT.at[c, pl.ds(r0, RW)], sem).wait()
        pltpu.make_async_copy(
            bits_v.at[b], bitsT.at[:, c, pl.ds(r0, RW)], sem).wait()
        pltpu.make_async_copy(
            emb_v.at[b], embT.at[c, :, pl.ds(r0, RW)], sem).wait()


@jax.jit
def _featurize(flagsT, tableT):
    mesh = plsc.VectorSubcoreMesh(core_axis_name="c", subcore_axis_name="s")
    fn = functools.partial(
        pl.kernel,
        out_type=(
            jax.ShapeDtypeStruct((COLS, ROWS), jnp.float32),
            jax.ShapeDtypeStruct((COLS, EMBED_DIM, ROWS), jnp.float32),
            jax.ShapeDtypeStruct((NUM_BITS, COLS, ROWS), jnp.float32),
        ),
        mesh=mesh,
        compiler_params=pltpu.CompilerParams(
            needs_layout_passes=False, use_tc_tiling_on_sc=False),
        scratch_types=[
            pltpu.VMEM((VOCAB * EMBED_DIM,), jnp.float32),
            pltpu.VMEM((2, CC, RW), jnp.int32),
            pltpu.VMEM((2, EMBED_DIM, RW), jnp.float32),
            pltpu.VMEM((2, NUM_BITS, RW), jnp.float32),
            pltpu.VMEM((2, RW), jnp.float32),
            pltpu.SemaphoreType.DMA,
            pltpu.SemaphoreType.DMA,
            pltpu.SemaphoreType.DMA,
        ],
    )(_sc_body)
    return fn(flagsT, tableT)


def kernel(qc_flags, table):
    flagsT = qc_flags.astype(jnp.int32).T
    tableT = table.T.reshape(-1)
    validT, embT, bitsT = _featurize(flagsT, tableT)
    return (
        validT.T,
        embT.transpose(2, 0, 1),
        bitsT.transpose(2, 1, 0),
    )


# speedup vs baseline: 21.1648x; 1.5269x over previous
import functools

import jax
import jax.numpy as jnp
from jax import lax
from jax.experimental import pallas as pl
from jax.experimental.pallas import tpu as pltpu
from jax.experimental.pallas import tpu_sc as plsc

NUM_BITS = 10
EMBED_DIM = 32
ROWS = 16384
COLS = 100
VOCAB = 1024

NC = 2
NS = 16
L = 16
NW = NC * NS
RW = ROWS // NW
CC = 20
NCH = COLS // CC


def _sc_body(flagsT, tableT, validT, embT, bitsT,
             table_v, flags_v, emb_v, bits_v, valid_v,
             in_sem, out_sem0, out_sem1):
    wid = lax.axis_index("s") * NC + lax.axis_index("c")
    r0 = wid * RW
    out_sems = (out_sem0, out_sem1)

    pltpu.sync_copy(tableT, table_v)
    pltpu.async_copy(
        flagsT.at[pl.ds(0, CC), pl.ds(r0, RW)], flags_v.at[0], in_sem
    ).wait()

    for k in range(NCH):
        in_slot = k % 2
        if k + 1 < NCH:
            pltpu.async_copy(
                flagsT.at[pl.ds((k + 1) * CC, CC), pl.ds(r0, RW)],
                flags_v.at[(k + 1) % 2], in_sem)

        @pl.loop(0, CC, step=2)
        def _col_pair(c2, _k=k, _slot=in_slot):
            for b in range(2):
                cl = c2 + b
                c = _k * CC + cl
                sem = out_sems[b]

                def drain():
                    pltpu.make_async_copy(
                        valid_v.at[b], validT.at[c, pl.ds(r0, RW)], sem
                    ).wait()
                    pltpu.make_async_copy(
                        bits_v.at[b], bitsT.at[:, c, pl.ds(r0, RW)], sem
                    ).wait()
                    pltpu.make_async_copy(
                        emb_v.at[b], embT.at[c, :, pl.ds(r0, RW)], sem
                    ).wait()

                if _k == 0:
                    @pl.when(c2 + b >= 2)
                    def _():
                        drain()
                else:
                    drain()

                @plsc.parallel_loop(0, RW // L, 1, unroll=4)
                def _vec(vi):
                    f = flags_v[_slot, cl, pl.ds(vi * L, L)]
                    valid_v[b, pl.ds(vi * L, L)] = jnp.where(
                        f == 0, jnp.float32(1.0), jnp.float32(0.0))
                    for t in range(NUM_BITS):
                        bits_v[b, t, pl.ds(vi * L, L)] = (
                            (f >> t) & 1).astype(jnp.float32)
                    for d in range(EMBED_DIM):
                        emb_v[b, d, pl.ds(vi * L, L)] = plsc.load_gather(
                            table_v, [f + d * VOCAB])

                pltpu.async_copy(
                    valid_v.at[b], validT.at[c, pl.ds(r0, RW)], sem)
                pltpu.async_copy(
                    bits_v.at[b], bitsT.at[:, c, pl.ds(r0, RW)], sem)
                pltpu.async_copy(
                    emb_v.at[b], embT.at[c, :, pl.ds(r0, RW)], sem)

        if k > 0 or True:
            pass

        if k + 1 < NCH:
            pltpu.make_async_copy(
                flagsT.at[pl.ds((k + 1) * CC, CC), pl.ds(r0, RW)],
                flags_v.at[(k + 1) % 2], in_sem
            ).wait()

    for b in range(2):
        c = COLS - 2 + b
        sem = out_sems[b]
        pltpu.make_async_copy(
            valid_v.at[b], validT.at[c, pl.ds(r0, RW)], sem).wait()
        pltpu.make_async_copy(
            bits_v.at[b], bitsT.at[:, c, pl.ds(r0, RW)], sem).wait()
        pltpu.make_async_copy(
            emb_v.at[b], embT.at[c, :, pl.ds(r0, RW)], sem).wait()


@jax.jit
def _featurize(flagsT, tableT):
    mesh = plsc.VectorSubcoreMesh(core_axis_name="c", subcore_axis_name="s")
    fn = functools.partial(
        pl.kernel,
        out_type=(
            jax.ShapeDtypeStruct((COLS, ROWS), jnp.float32),
            jax.ShapeDtypeStruct((COLS, EMBED_DIM, ROWS), jnp.float32),
            jax.ShapeDtypeStruct((NUM_BITS, COLS, ROWS), jnp.float32),
        ),
        mesh=mesh,
        compiler_params=pltpu.CompilerParams(
            needs_layout_passes=False, use_tc_tiling_on_sc=False),
        scratch_types=[
            pltpu.VMEM((VOCAB * EMBED_DIM,), jnp.float32),
            pltpu.VMEM((2, CC, RW), jnp.int32),
            pltpu.VMEM((2, EMBED_DIM, RW), jnp.float32),
            pltpu.VMEM((2, NUM_BITS, RW), jnp.float32),
            pltpu.VMEM((2, RW), jnp.float32),
            pltpu.SemaphoreType.DMA,
            pltpu.SemaphoreType.DMA,
            pltpu.SemaphoreType.DMA,
        ],
    )(_sc_body)
    return fn(flagsT, tableT)


def kernel(qc_flags, table):
    flagsT = qc_flags.astype(jnp.int32).T
    tableT = table.T.reshape(-1)
    validT, embT, bitsT = _featurize(flagsT, tableT)
    return (
        validT.T,
        embT.transpose(2, 0, 1),
        bitsT.transpose(2, 1, 0),
    )


# speedup vs baseline: 41.6384x; 1.9673x over previous
import functools

import jax
import jax.numpy as jnp
from jax import lax
from jax.experimental import pallas as pl
from jax.experimental.pallas import tpu as pltpu
from jax.experimental.pallas import tpu_sc as plsc

NUM_BITS = 10
EMBED_DIM = 32
ROWS = 16384
COLS = 100
VOCAB = 1024

NC = 2
NS = 16
L = 16
NW = NC * NS
RW = ROWS // NW
CC = 20
NCH = COLS // CC


def _sc_body(flagsT, tableT, validT, embT, bitsT,
             table_v, flags_v, emb_v, bits_v, valid_v,
             in_sem, out_sem0, out_sem1):
    wid = lax.axis_index("s") * NC + lax.axis_index("c")
    r0 = wid * RW
    out_sems = (out_sem0, out_sem1)

    pltpu.sync_copy(tableT, table_v)
    pltpu.async_copy(
        flagsT.at[pl.ds(0, CC), pl.ds(r0, RW)], flags_v.at[0], in_sem
    ).wait()

    for k in range(NCH):
        in_slot = k % 2
        if k + 1 < NCH:
            pltpu.async_copy(
                flagsT.at[pl.ds((k + 1) * CC, CC), pl.ds(r0, RW)],
                flags_v.at[(k + 1) % 2], in_sem)

        @pl.loop(0, CC, step=2)
        def _col_pair(c2, _k=k, _slot=in_slot):
            for b in range(2):
                cl = c2 + b
                c = _k * CC + cl
                sem = out_sems[b]

                def drain():
                    pltpu.make_async_copy(
                        valid_v.at[b], validT.at[c, pl.ds(r0, RW)], sem
                    ).wait()
                    pltpu.make_async_copy(
                        bits_v.at[b], bitsT.at[:, c, pl.ds(r0, RW)], sem
                    ).wait()
                    pltpu.make_async_copy(
                        emb_v.at[b],
                        embT.at[c, :, pl.ds(wid * (RW // 128), RW // 128)],
                        sem).wait()

                if _k == 0:
                    @pl.when(c2 + b >= 2)
                    def _():
                        drain()
                else:
                    drain()

                @plsc.parallel_loop(0, RW // L, 1, unroll=4)
                def _vec(vi):
                    f = flags_v[_slot, cl, pl.ds(vi * L, L)]
                    valid_v[b, pl.ds(vi * L, L)] = jnp.where(
                        f == 0, jnp.float32(1.0), jnp.float32(0.0))
                    for t in range(NUM_BITS):
                        bits_v[b, t, pl.ds(vi * L, L)] = (
                            (f >> t) & 1).astype(jnp.float32)
                    for d in range(EMBED_DIM):
                        emb_v[b, d // 8, vi // 8, d % 8,
                              pl.ds((vi % 8) * L, L)] = plsc.load_gather(
                            table_v, [f + d * VOCAB])

                pltpu.async_copy(
                    valid_v.at[b], validT.at[c, pl.ds(r0, RW)], sem)
                pltpu.async_copy(
                    bits_v.at[b], bitsT.at[:, c, pl.ds(r0, RW)], sem)
                pltpu.async_copy(
                    emb_v.at[b],
                    embT.at[c, :, pl.ds(wid * (RW // 128), RW // 128)], sem)

        if k > 0 or True:
            pass

        if k + 1 < NCH:
            pltpu.make_async_copy(
                flagsT.at[pl.ds((k + 1) * CC, CC), pl.ds(r0, RW)],
                flags_v.at[(k + 1) % 2], in_sem
            ).wait()

    for b in range(2):
        c = COLS - 2 + b
        sem = out_sems[b]
        pltpu.make_async_copy(
            valid_v.at[b], validT.at[c, pl.ds(r0, RW)], sem).wait()
        pltpu.make_async_copy(
            bits_v.at[b], bitsT.at[:, c, pl.ds(r0, RW)], sem).wait()
        pltpu.make_async_copy(
            emb_v.at[b],
            embT.at[c, :, pl.ds(wid * (RW // 128), RW // 128)], sem).wait()


@jax.jit
def _featurize(flagsT, tableT):
    mesh = plsc.VectorSubcoreMesh(core_axis_name="c", subcore_axis_name="s")
    fn = functools.partial(
        pl.kernel,
        out_type=(
            jax.ShapeDtypeStruct((COLS, ROWS), jnp.float32),
            jax.ShapeDtypeStruct(
                (COLS, EMBED_DIM // 8, ROWS // 128, 8, 128), jnp.float32),
            jax.ShapeDtypeStruct((NUM_BITS, COLS, ROWS), jnp.float32),
        ),
        mesh=mesh,
        compiler_params=pltpu.CompilerParams(
            needs_layout_passes=False, use_tc_tiling_on_sc=False),
        scratch_types=[
            pltpu.VMEM((VOCAB * EMBED_DIM,), jnp.float32),
            pltpu.VMEM((2, CC, RW), jnp.int32),
            pltpu.VMEM((2, EMBED_DIM // 8, RW // 128, 8, 128), jnp.float32),
            pltpu.VMEM((2, NUM_BITS, RW), jnp.float32),
            pltpu.VMEM((2, RW), jnp.float32),
            pltpu.SemaphoreType.DMA,
            pltpu.SemaphoreType.DMA,
            pltpu.SemaphoreType.DMA,
        ],
    )(_sc_body)
    return fn(flagsT, tableT)


def kernel(qc_flags, table):
    flagsT = qc_flags.astype(jnp.int32).T
    tableT = table.T.reshape(-1)
    validT, embT, bitsT = _featurize(flagsT, tableT)
    emb = embT.transpose(2, 4, 0, 1, 3).reshape(ROWS, COLS, EMBED_DIM)
    return (
        validT.T,
        emb,
        bitsT.transpose(2, 1, 0),
    )
